# baseline (speedup 1.0000x reference)
import functools

import jax
import jax.numpy as jnp
from jax import lax
from jax.experimental import pallas as pl
from jax.experimental.pallas import tpu as pltpu
from jax.experimental.pallas import tpu_sc as plsc

N_NODES = 10000
D = 128
DH = D // 2
N_EDGES = 320000

NC = 2
NS = 16
EPT = N_EDGES // NS
CHUNK = 160
NG = EPT // CHUNK
NSTAGE = 5
HG = NG // NSTAGE
ROWS_PER_TILE = 624
TAIL_ROWS = N_NODES - NS * ROWS_PER_TILE
TAIL_OFF = NS * ROWS_PER_TILE

_mesh = plsc.VectorSubcoreMesh(core_axis_name="c", subcore_axis_name="s",
                               num_cores=NC, num_subcores=NS)


@functools.partial(
    pl.kernel,
    out_type=jax.ShapeDtypeStruct((NC, N_NODES, DH), jnp.float32),
    mesh=_mesh,
    scratch_types=[
        pltpu.VMEM((HG, CHUNK), jnp.int32),
        pltpu.VMEM((HG, CHUNK), jnp.int32),
        pltpu.VMEM((CHUNK, DH), jnp.float32),
        pltpu.VMEM((CHUNK, DH), jnp.float32),
        pltpu.VMEM_SHARED((N_NODES, DH), jnp.float32),
        pltpu.VMEM_SHARED((N_NODES, DH), jnp.float32),
        pltpu.SemaphoreType.DMA,
        pltpu.SemaphoreType.DMA,
    ],
    compiler_params=pltpu.CompilerParams(use_tc_tiling_on_sc=False),
)
def _sc_aggregate(xt_hbm, src_hbm, dst_hbm, out_hbm,
                  src_v, dst_v, rows_a, rows_b, agg_sh, xsh, sem_a, sem_b):
    c = lax.axis_index("c")
    s = lax.axis_index("s")
    x_half = xt_hbm.at[c]

    r0 = s * ROWS_PER_TILE
    pltpu.sync_copy(x_half.at[pl.ds(r0, ROWS_PER_TILE)],
                    agg_sh.at[pl.ds(r0, ROWS_PER_TILE)])
    pltpu.sync_copy(x_half.at[pl.ds(r0, ROWS_PER_TILE)],
                    xsh.at[pl.ds(r0, ROWS_PER_TILE)])

    @pl.when(s == NS - 1)
    def _():
        pltpu.sync_copy(x_half.at[pl.ds(TAIL_OFF, TAIL_ROWS)],
                        agg_sh.at[pl.ds(TAIL_OFF, TAIL_ROWS)])
        pltpu.sync_copy(x_half.at[pl.ds(TAIL_OFF, TAIL_ROWS)],
                        xsh.at[pl.ds(TAIL_OFF, TAIL_ROWS)])

    plsc.subcore_barrier()

    for h in (0, 1, 2, 3, 4):
        pltpu.sync_copy(src_hbm.at[s].at[pl.ds(h * HG, HG)], src_v)
        pltpu.sync_copy(dst_hbm.at[s].at[pl.ds(h * HG, HG)], dst_v)
        pltpu.async_copy(xsh.at[src_v.at[0]], rows_a, sem_a)

        def two_groups(t, carry):
            l0 = 2 * t
            l1 = l0 + 1
            pltpu.async_copy(xsh.at[src_v.at[l1]], rows_b, sem_b)
            pltpu.make_async_copy(xsh.at[src_v.at[0]], rows_a, sem_a).wait()
            pltpu.sync_copy(rows_a, agg_sh.at[dst_v.at[l0]], add=True)
            pltpu.async_copy(xsh.at[src_v.at[l0 + 2]], rows_a, sem_a)
            pltpu.make_async_copy(xsh.at[src_v.at[0]], rows_b, sem_b).wait()
            pltpu.sync_copy(rows_b, agg_sh.at[dst_v.at[l1]], add=True)
            return carry

        lax.fori_loop(0, HG // 2, two_groups, 0)
        pltpu.make_async_copy(xsh.at[src_v.at[0]], rows_a, sem_a).wait()
        pltpu.sync_copy(rows_a, agg_sh.at[dst_v.at[HG - 1]], add=True)

    plsc.subcore_barrier()

    out_half = out_hbm.at[c]
    pltpu.sync_copy(agg_sh.at[pl.ds(r0, ROWS_PER_TILE)],
                    out_half.at[pl.ds(r0, ROWS_PER_TILE)])

    @pl.when(s == NS - 1)
    def _():
        pltpu.sync_copy(agg_sh.at[pl.ds(TAIL_OFF, TAIL_ROWS)],
                        out_half.at[pl.ds(TAIL_OFF, TAIL_ROWS)])


BLK = 1000


def _mlp_body(p_ref, w1_ref, b1_ref, w2_ref, b2_ref, o_ref):
    h = jnp.concatenate([p_ref[0], p_ref[1]], axis=-1)
    h = jnp.dot(h, w1_ref[...], preferred_element_type=jnp.float32) + b1_ref[...]
    h = jnp.maximum(h, 0.0)
    o_ref[...] = jnp.dot(h, w2_ref[...], preferred_element_type=jnp.float32) + b2_ref[...]


_mlp = pl.pallas_call(
    _mlp_body,
    grid=(N_NODES // BLK,),
    in_specs=[
        pl.BlockSpec((NC, BLK, DH), lambda i: (0, i, 0)),
        pl.BlockSpec((D, D), lambda i: (0, 0)),
        pl.BlockSpec((1, D), lambda i: (0, 0)),
        pl.BlockSpec((D, D), lambda i: (0, 0)),
        pl.BlockSpec((1, D), lambda i: (0, 0)),
    ],
    out_specs=pl.BlockSpec((BLK, D), lambda i: (i, 0)),
    out_shape=jax.ShapeDtypeStruct((N_NODES, D), jnp.float32),
)


def kernel(x, edge_index, W1, b1, W2, b2):
    xt = x.reshape(N_NODES, NC, DH).transpose(1, 0, 2)
    src = edge_index[0].astype(jnp.int32).reshape(NS, NG, CHUNK)
    dst = edge_index[1].astype(jnp.int32).reshape(NS, NG, CHUNK)
    p = _sc_aggregate(xt, src, dst)
    return _mlp(p, W1, b1.reshape(1, D), W2, b2.reshape(1, D))


# speedup vs baseline: 1.2934x; 1.2934x over previous
import functools

import jax
import jax.numpy as jnp
from jax import lax
from jax.experimental import pallas as pl
from jax.experimental.pallas import tpu as pltpu
from jax.experimental.pallas import tpu_sc as plsc

N_NODES = 10000
D = 128
DH = D // 2
N_EDGES = 320000

NC = 2
NS = 16
EPT = N_EDGES // NS
CHUNK = 200
NG = EPT // CHUNK
HG = NG // 2
NB = 5
ROWS_PER_TILE = 624
TAIL_ROWS = N_NODES - NS * ROWS_PER_TILE
TAIL_OFF = NS * ROWS_PER_TILE

_mesh = plsc.VectorSubcoreMesh(core_axis_name="c", subcore_axis_name="s",
                               num_cores=NC, num_subcores=NS)


@functools.partial(
    pl.kernel,
    out_type=jax.ShapeDtypeStruct((NC, N_NODES, DH), jnp.float32),
    mesh=_mesh,
    scratch_types=[
        pltpu.VMEM((HG, CHUNK), jnp.int32),
        pltpu.VMEM((HG, CHUNK), jnp.int32),
        [pltpu.VMEM((CHUNK, DH), jnp.float32) for _ in range(NB)],
        pltpu.VMEM_SHARED((N_NODES, DH), jnp.float32),
        [pltpu.SemaphoreType.DMA for _ in range(NB)],
        [pltpu.SemaphoreType.DMA for _ in range(NB)],
    ],
    compiler_params=pltpu.CompilerParams(use_tc_tiling_on_sc=False),
)
def _sc_aggregate(xt_hbm, src_hbm, dst_hbm, out_hbm,
                  src_v, dst_v, rows, agg_sh, gsem, ssem):
    c = lax.axis_index("c")
    s = lax.axis_index("s")
    x_half = xt_hbm.at[c]

    r0 = s * ROWS_PER_TILE
    pltpu.sync_copy(x_half.at[pl.ds(r0, ROWS_PER_TILE)],
                    agg_sh.at[pl.ds(r0, ROWS_PER_TILE)])

    @pl.when(s == NS - 1)
    def _():
        pltpu.sync_copy(x_half.at[pl.ds(TAIL_OFF, TAIL_ROWS)],
                        agg_sh.at[pl.ds(TAIL_OFF, TAIL_ROWS)])

    plsc.subcore_barrier()

    NR = HG // NB
    for h in (0, 1):
        pltpu.sync_copy(src_hbm.at[s].at[pl.ds(h * HG, HG)], src_v)
        pltpu.sync_copy(dst_hbm.at[s].at[pl.ds(h * HG, HG)], dst_v)
        for b in range(NB):
            pltpu.async_copy(x_half.at[src_v.at[b]], rows[b], gsem[b])

        def round_(r, carry):
            g0 = NB * r
            for b in range(NB):
                pltpu.make_async_copy(x_half.at[src_v.at[0]],
                                      rows[b], gsem[b]).wait()
                pltpu.async_copy(rows[b], agg_sh.at[dst_v.at[g0 + b]],
                                 ssem[b], add=True)
            for b in range(NB):
                pltpu.make_async_copy(rows[b], agg_sh.at[dst_v.at[0]],
                                      ssem[b]).wait()

                @pl.when(r < NR - 1)
                def _():
                    pltpu.async_copy(x_half.at[src_v.at[g0 + NB + b]],
                                     rows[b], gsem[b])
            return carry

        lax.fori_loop(0, NR, round_, 0)

    plsc.subcore_barrier()

    out_half = out_hbm.at[c]
    pltpu.sync_copy(agg_sh.at[pl.ds(r0, ROWS_PER_TILE)],
                    out_half.at[pl.ds(r0, ROWS_PER_TILE)])

    @pl.when(s == NS - 1)
    def _():
        pltpu.sync_copy(agg_sh.at[pl.ds(TAIL_OFF, TAIL_ROWS)],
                        out_half.at[pl.ds(TAIL_OFF, TAIL_ROWS)])


BLK = 1000


def _mlp_body(p_ref, w1_ref, b1_ref, w2_ref, b2_ref, o_ref):
    h = jnp.concatenate([p_ref[0], p_ref[1]], axis=-1)
    h = jnp.dot(h, w1_ref[...], preferred_element_type=jnp.float32) + b1_ref[...]
    h = jnp.maximum(h, 0.0)
    o_ref[...] = jnp.dot(h, w2_ref[...], preferred_element_type=jnp.float32) + b2_ref[...]


_mlp = pl.pallas_call(
    _mlp_body,
    grid=(N_NODES // BLK,),
    in_specs=[
        pl.BlockSpec((NC, BLK, DH), lambda i: (0, i, 0)),
        pl.BlockSpec((D, D), lambda i: (0, 0)),
        pl.BlockSpec((1, D), lambda i: (0, 0)),
        pl.BlockSpec((D, D), lambda i: (0, 0)),
        pl.BlockSpec((1, D), lambda i: (0, 0)),
    ],
    out_specs=pl.BlockSpec((BLK, D), lambda i: (i, 0)),
    out_shape=jax.ShapeDtypeStruct((N_NODES, D), jnp.float32),
)


def kernel(x, edge_index, W1, b1, W2, b2):
    xt = x.reshape(N_NODES, NC, DH).transpose(1, 0, 2)
    src = edge_index[0].astype(jnp.int32).reshape(NS, NG, CHUNK)
    dst = edge_index[1].astype(jnp.int32).reshape(NS, NG, CHUNK)
    p = _sc_aggregate(xt, src, dst)
    return _mlp(p, W1, b1.reshape(1, D), W2, b2.reshape(1, D))


# speedup vs baseline: 1.3879x; 1.0731x over previous
import functools

import jax
import jax.numpy as jnp
from jax import lax
from jax.experimental import pallas as pl
from jax.experimental.pallas import tpu as pltpu
from jax.experimental.pallas import tpu_sc as plsc

N_NODES = 10000
D = 128
DH = D // 2
N_EDGES = 320000

NC = 2
NS = 16
EPT = N_EDGES // NS
CHUNK = 400
NG = EPT // CHUNK
HG = NG // 2
ROWS_PER_TILE = 624
TAIL_ROWS = N_NODES - NS * ROWS_PER_TILE
TAIL_OFF = NS * ROWS_PER_TILE

_mesh = plsc.VectorSubcoreMesh(core_axis_name="c", subcore_axis_name="s",
                               num_cores=NC, num_subcores=NS)


@functools.partial(
    pl.kernel,
    out_type=jax.ShapeDtypeStruct((NC, N_NODES, DH), jnp.float32),
    mesh=_mesh,
    scratch_types=[
        pltpu.VMEM((HG, CHUNK), jnp.int32),
        pltpu.VMEM((HG, CHUNK), jnp.int32),
        pltpu.VMEM((CHUNK, DH), jnp.float32),
        pltpu.VMEM((CHUNK, DH), jnp.float32),
        pltpu.VMEM_SHARED((N_NODES, DH), jnp.float32),
        pltpu.SemaphoreType.DMA,
        pltpu.SemaphoreType.DMA,
    ],
    compiler_params=pltpu.CompilerParams(use_tc_tiling_on_sc=False),
)
def _sc_aggregate(x2_hbm, z_hbm, src_hbm, dst_hbm, out_hbm,
                  src_v, dst_v, rows_a, rows_b, agg_sh, sem_a, sem_b):
    c = lax.axis_index("c")
    s = lax.axis_index("s")
    x_half = x2_hbm

    r0 = s * ROWS_PER_TILE
    pltpu.sync_copy(z_hbm, agg_sh.at[pl.ds(r0, ROWS_PER_TILE)])

    @pl.when(s == NS - 1)
    def _():
        pltpu.sync_copy(z_hbm.at[pl.ds(0, TAIL_ROWS)],
                        agg_sh.at[pl.ds(TAIL_OFF, TAIL_ROWS)])

    plsc.subcore_barrier()

    for h in (0, 1):
        pltpu.sync_copy(src_hbm.at[c].at[s].at[pl.ds(h * HG, HG)], src_v)
        pltpu.sync_copy(dst_hbm.at[s].at[pl.ds(h * HG, HG)], dst_v)
        pltpu.async_copy(x_half.at[src_v.at[0]], rows_a, sem_a)

        def two_groups(t, carry):
            l0 = 2 * t
            l1 = l0 + 1
            pltpu.async_copy(x_half.at[src_v.at[l1]], rows_b, sem_b)
            pltpu.make_async_copy(x_half.at[src_v.at[0]], rows_a, sem_a).wait()
            pltpu.sync_copy(rows_a, agg_sh.at[dst_v.at[l0]], add=True)
            pltpu.async_copy(x_half.at[src_v.at[l0 + 2]], rows_a, sem_a)
            pltpu.make_async_copy(x_half.at[src_v.at[0]], rows_b, sem_b).wait()
            pltpu.sync_copy(rows_b, agg_sh.at[dst_v.at[l1]], add=True)
            return carry

        lax.fori_loop(0, HG // 2, two_groups, 0)
        pltpu.make_async_copy(x_half.at[src_v.at[0]], rows_a, sem_a).wait()
        pltpu.sync_copy(rows_a, agg_sh.at[dst_v.at[HG - 1]], add=True)

    plsc.subcore_barrier()

    out_half = out_hbm.at[c]
    pltpu.sync_copy(agg_sh.at[pl.ds(r0, ROWS_PER_TILE)],
                    out_half.at[pl.ds(r0, ROWS_PER_TILE)])

    @pl.when(s == NS - 1)
    def _():
        pltpu.sync_copy(agg_sh.at[pl.ds(TAIL_OFF, TAIL_ROWS)],
                        out_half.at[pl.ds(TAIL_OFF, TAIL_ROWS)])


BLK = 1000


def _mlp_body(p_ref, x_ref, w1_ref, b1_ref, w2_ref, b2_ref, o_ref):
    h = jnp.concatenate([p_ref[0], p_ref[1]], axis=-1) + x_ref[...]
    h = jnp.dot(h, w1_ref[...], preferred_element_type=jnp.float32) + b1_ref[...]
    h = jnp.maximum(h, 0.0)
    o_ref[...] = jnp.dot(h, w2_ref[...], preferred_element_type=jnp.float32) + b2_ref[...]


_mlp = pl.pallas_call(
    _mlp_body,
    grid=(N_NODES // BLK,),
    in_specs=[
        pl.BlockSpec((NC, BLK, DH), lambda i: (0, i, 0)),
        pl.BlockSpec((BLK, D), lambda i: (i, 0)),
        pl.BlockSpec((D, D), lambda i: (0, 0)),
        pl.BlockSpec((1, D), lambda i: (0, 0)),
        pl.BlockSpec((D, D), lambda i: (0, 0)),
        pl.BlockSpec((1, D), lambda i: (0, 0)),
    ],
    out_specs=pl.BlockSpec((BLK, D), lambda i: (i, 0)),
    out_shape=jax.ShapeDtypeStruct((N_NODES, D), jnp.float32),
)


def kernel(x, edge_index, W1, b1, W2, b2):
    x2 = x.reshape(NC * N_NODES, DH)
    src = edge_index[0].astype(jnp.int32).reshape(1, NS, NG, CHUNK)
    srcd = jnp.concatenate([2 * src, 2 * src + 1], axis=0)
    dst = edge_index[1].astype(jnp.int32).reshape(NS, NG, CHUNK)
    z = jnp.zeros((ROWS_PER_TILE, DH), jnp.float32)
    p = _sc_aggregate(x2, z, srcd, dst)
    return _mlp(p, x, W1, b1.reshape(1, D), W2, b2.reshape(1, D))


# speedup vs baseline: 1.4275x; 1.0285x over previous
import functools

import jax
import jax.numpy as jnp
from jax import lax
from jax.experimental import pallas as pl
from jax.experimental.pallas import tpu as pltpu
from jax.experimental.pallas import tpu_sc as plsc

N_NODES = 10000
D = 128
DH = D // 2
N_EDGES = 320000

NC = 2
NS = 16
EPT = N_EDGES // NS
CHUNK = 400
NG = EPT // CHUNK
HG = NG // 2
ROWS_PER_TILE = 624
TAIL_ROWS = N_NODES - NS * ROWS_PER_TILE
TAIL_OFF = NS * ROWS_PER_TILE

_mesh = plsc.VectorSubcoreMesh(core_axis_name="c", subcore_axis_name="s",
                               num_cores=NC, num_subcores=NS)


@functools.partial(
    pl.kernel,
    out_type=jax.ShapeDtypeStruct((NC, N_NODES, DH), jnp.float32),
    mesh=_mesh,
    scratch_types=[
        pltpu.VMEM((HG, CHUNK), jnp.int32),
        pltpu.VMEM((HG, CHUNK), jnp.int32),
        pltpu.VMEM((CHUNK, DH), jnp.float32),
        pltpu.VMEM((CHUNK, DH), jnp.float32),
        pltpu.VMEM_SHARED((N_NODES, DH), jnp.float32),
        pltpu.SemaphoreType.DMA,
        pltpu.SemaphoreType.DMA,
    ],
    compiler_params=pltpu.CompilerParams(use_tc_tiling_on_sc=False),
)
def _sc_aggregate(x2_hbm, z_hbm, src_hbm, dst_hbm, out_hbm,
                  src_v, dst_v, rows_a, rows_b, agg_sh, sem_a, sem_b):
    c = lax.axis_index("c")
    s = lax.axis_index("s")
    x_half = x2_hbm.at[pl.ds(c, NC * N_NODES - 1)]

    r0 = s * ROWS_PER_TILE
    pltpu.sync_copy(z_hbm, agg_sh.at[pl.ds(r0, ROWS_PER_TILE)])

    @pl.when(s == NS - 1)
    def _():
        pltpu.sync_copy(z_hbm.at[pl.ds(0, TAIL_ROWS)],
                        agg_sh.at[pl.ds(TAIL_OFF, TAIL_ROWS)])

    plsc.subcore_barrier()

    for h in (0, 1):
        pltpu.sync_copy(src_hbm.at[s].at[pl.ds(h * HG, HG)], src_v)
        pltpu.sync_copy(dst_hbm.at[s].at[pl.ds(h * HG, HG)], dst_v)
        pltpu.async_copy(x_half.at[src_v.at[0]], rows_a, sem_a)

        def two_groups(t, carry):
            l0 = 2 * t
            l1 = l0 + 1
            pltpu.async_copy(x_half.at[src_v.at[l1]], rows_b, sem_b)
            pltpu.make_async_copy(x_half.at[src_v.at[0]], rows_a, sem_a).wait()
            pltpu.sync_copy(rows_a, agg_sh.at[dst_v.at[l0]], add=True)
            pltpu.async_copy(x_half.at[src_v.at[l0 + 2]], rows_a, sem_a)
            pltpu.make_async_copy(x_half.at[src_v.at[0]], rows_b, sem_b).wait()
            pltpu.sync_copy(rows_b, agg_sh.at[dst_v.at[l1]], add=True)
            return carry

        lax.fori_loop(0, HG // 2, two_groups, 0)
        pltpu.make_async_copy(x_half.at[src_v.at[0]], rows_a, sem_a).wait()
        pltpu.sync_copy(rows_a, agg_sh.at[dst_v.at[HG - 1]], add=True)

    plsc.subcore_barrier()

    out_half = out_hbm.at[c]
    pltpu.sync_copy(agg_sh.at[pl.ds(r0, ROWS_PER_TILE)],
                    out_half.at[pl.ds(r0, ROWS_PER_TILE)])

    @pl.when(s == NS - 1)
    def _():
        pltpu.sync_copy(agg_sh.at[pl.ds(TAIL_OFF, TAIL_ROWS)],
                        out_half.at[pl.ds(TAIL_OFF, TAIL_ROWS)])


BLK = 1000


def _mlp_body(p_ref, x_ref, w1_ref, b1_ref, w2_ref, b2_ref, o_ref):
    h = jnp.concatenate([p_ref[0], p_ref[1]], axis=-1) + x_ref[...]
    h = jnp.dot(h, w1_ref[...], preferred_element_type=jnp.float32) + b1_ref[...]
    h = jnp.maximum(h, 0.0)
    o_ref[...] = jnp.dot(h, w2_ref[...], preferred_element_type=jnp.float32) + b2_ref[...]


_mlp = pl.pallas_call(
    _mlp_body,
    grid=(N_NODES // BLK,),
    in_specs=[
        pl.BlockSpec((NC, BLK, DH), lambda i: (0, i, 0)),
        pl.BlockSpec((BLK, D), lambda i: (i, 0)),
        pl.BlockSpec((D, D), lambda i: (0, 0)),
        pl.BlockSpec((1, D), lambda i: (0, 0)),
        pl.BlockSpec((D, D), lambda i: (0, 0)),
        pl.BlockSpec((1, D), lambda i: (0, 0)),
    ],
    out_specs=pl.BlockSpec((BLK, D), lambda i: (i, 0)),
    out_shape=jax.ShapeDtypeStruct((N_NODES, D), jnp.float32),
)


def kernel(x, edge_index, W1, b1, W2, b2):
    x2 = x.reshape(NC * N_NODES, DH)
    srcd = (2 * edge_index[0].astype(jnp.int32)).reshape(NS, NG, CHUNK)
    dst = edge_index[1].astype(jnp.int32).reshape(NS, NG, CHUNK)
    z = jnp.zeros((ROWS_PER_TILE, DH), jnp.float32)
    p = _sc_aggregate(x2, z, srcd, dst)
    return _mlp(p, x, W1, b1.reshape(1, D), W2, b2.reshape(1, D))
